# baseline (speedup 1.0000x reference)
import functools

import jax
import jax.numpy as jnp
from jax import lax
from jax.experimental import pallas as pl
from jax.experimental.pallas import tpu as pltpu
from jax.experimental.pallas import tpu_sc as plsc

_INV_T = 10.0
_NEG_INIT = -1e30


def _sc_rowstats(neg, row_off, n_sc, chunk, unroll):
    _, m = neg.shape
    info = plsc.get_sparse_core_info()
    ncores, nsub, lanes = info.num_cores, info.num_subcores, info.num_lanes
    nw = ncores * nsub
    rows_w = n_sc // nw
    assert rows_w * nw == n_sc and m // chunk == 2 and m % chunk == 0
    nacc = 4

    mesh = plsc.VectorSubcoreMesh(core_axis_name="c", subcore_axis_name="s")

    @functools.partial(
        pl.kernel,
        mesh=mesh,
        out_type=(
            jax.ShapeDtypeStruct((n_sc, lanes), jnp.float32),
            jax.ShapeDtypeStruct((n_sc, lanes), jnp.float32),
        ),
        scratch_types=[
            pltpu.VMEM((chunk,), jnp.float32),
            pltpu.VMEM((chunk,), jnp.float32),
            pltpu.VMEM((rows_w, lanes), jnp.float32),
            pltpu.VMEM((rows_w, lanes), jnp.float32),
            pltpu.SemaphoreType.DMA,
            pltpu.SemaphoreType.DMA,
        ],
    )
    def k(neg_hbm, mv_hbm, sv_hbm, buf0, buf1, mrow, srow, sem0, sem1):
        wid = lax.axis_index("s") * ncores + lax.axis_index("c")
        base = row_off + wid * rows_w

        def process(buf, mv, sv):
            @plsc.parallel_loop(
                0, chunk, step=unroll * lanes,
                carry=tuple(
                    jnp.full((lanes,), _NEG_INIT, jnp.float32) for _ in range(nacc)
                ),
            )
            def max_loop(off, cms):
                cms = list(cms)
                for u in range(unroll):
                    x = buf[pl.ds(off + u * lanes, lanes)]
                    cms[u % nacc] = jnp.maximum(cms[u % nacc], x)
                return tuple(cms)

            cm = max_loop[0]
            for a in range(1, nacc):
                cm = jnp.maximum(cm, max_loop[a])
            m_new = jnp.maximum(mv, cm * _INV_T)

            @plsc.parallel_loop(
                0, chunk, step=unroll * lanes,
                carry=tuple(jnp.zeros((lanes,), jnp.float32) for _ in range(nacc)),
            )
            def sum_loop(off, accs):
                accs = list(accs)
                for u in range(unroll):
                    x = buf[pl.ds(off + u * lanes, lanes)]
                    accs[u % nacc] = accs[u % nacc] + jnp.exp(x * _INV_T - m_new)
                return tuple(accs)

            se = sum_loop[0]
            for a in range(1, nacc):
                se = se + sum_loop[a]
            sv = sv * jnp.exp(mv - m_new) + se
            return m_new, sv

        pltpu.make_async_copy(
            neg_hbm.at[base, pl.ds(0, chunk)], buf0, sem0
        ).start()

        def row_body(r, carry):
            row = base + r
            pltpu.make_async_copy(
                neg_hbm.at[row, pl.ds(0, chunk)], buf0, sem0
            ).wait()
            pltpu.make_async_copy(
                neg_hbm.at[row, pl.ds(chunk, chunk)], buf1, sem1
            ).start()
            mv0 = jnp.full((lanes,), _NEG_INIT, jnp.float32)
            sv0 = jnp.zeros((lanes,), jnp.float32)
            mv, sv = process(buf0, mv0, sv0)
            pltpu.make_async_copy(
                neg_hbm.at[row, pl.ds(chunk, chunk)], buf1, sem1
            ).wait()

            @pl.when(r + 1 < rows_w)
            def _():
                pltpu.make_async_copy(
                    neg_hbm.at[row + 1, pl.ds(0, chunk)], buf0, sem0
                ).start()

            mv, sv = process(buf1, mv, sv)
            mrow[r, :] = mv
            srow[r, :] = sv
            return carry

        lax.fori_loop(0, rows_w, row_body, 0)
        pltpu.sync_copy(mrow, mv_hbm.at[pl.ds(wid * rows_w, rows_w)])
        pltpu.sync_copy(srow, sv_hbm.at[pl.ds(wid * rows_w, rows_w)])

    return k(neg)


def _tc_partial_kernel(pos_ref, neg_ref, out_ref, m_ref, s_ref, acc_ref, *, nr, nc):
    ri = pl.program_id(0)
    ci = pl.program_id(1)

    @pl.when(jnp.logical_and(ri == 0, ci == 0))
    def _():
        acc_ref[0, 0] = 0.0

    p = pos_ref[:, :] * _INV_T

    @pl.when(ci == 0)
    def _():
        m_ref[:, :] = p
        s_ref[:, :] = jnp.ones_like(p)

    blk = neg_ref[:, :] * _INV_T
    bm = jnp.max(blk, axis=1, keepdims=True)
    m_old = m_ref[:, :]
    m_new = jnp.maximum(m_old, bm)
    s_ref[:, :] = s_ref[:, :] * jnp.exp(m_old - m_new) + jnp.sum(
        jnp.exp(blk - m_new), axis=1, keepdims=True
    )
    m_ref[:, :] = m_new

    @pl.when(ci == nc - 1)
    def _():
        lse = m_ref[:, :] + jnp.log(s_ref[:, :])
        acc_ref[0, 0] += jnp.sum(lse - p)

        @pl.when(ri == nr - 1)
        def _():
            out_ref[:, :] = jnp.full((1, 1), acc_ref[0, 0], jnp.float32)


def _tc_partial(pos, neg, n_tc, br, bc):
    _, m = neg.shape
    nr = n_tc // br
    nc = m // bc
    return pl.pallas_call(
        functools.partial(_tc_partial_kernel, nr=nr, nc=nc),
        grid=(nr, nc),
        in_specs=[
            pl.BlockSpec((br, 1), lambda ri, ci: (ri, 0)),
            pl.BlockSpec((br, bc), lambda ri, ci: (ri, ci)),
        ],
        out_specs=pl.BlockSpec((1, 1), lambda ri, ci: (0, 0)),
        out_shape=jax.ShapeDtypeStruct((1, 1), jnp.float32),
        scratch_shapes=[
            pltpu.VMEM((br, 1), jnp.float32),
            pltpu.VMEM((br, 1), jnp.float32),
            pltpu.SMEM((1, 1), jnp.float32),
        ],
        compiler_params=pltpu.CompilerParams(
            dimension_semantics=("arbitrary", "arbitrary"),
        ),
    )(pos, neg)


def _finish_kernel(pos_ref, mv_ref, sv_ref, tcp_ref, out_ref, *, n_total):
    p = pos_ref[:, :] * _INV_T
    mv = mv_ref[:, :]
    sv = sv_ref[:, :]
    mt = jnp.maximum(jnp.max(mv, axis=1, keepdims=True), p)
    s = jnp.sum(sv * jnp.exp(mv - mt), axis=1, keepdims=True) + jnp.exp(p - mt)
    lse = mt + jnp.log(s)
    total = jnp.sum(lse - p) + tcp_ref[0, 0]
    out_ref[:, :] = jnp.full((1, 1), total / n_total, jnp.float32)


def kernel(pos, neg):
    n, m = neg.shape
    n_tc = 2304
    n_sc = n - n_tc
    mv, sv = _sc_rowstats(neg, n_tc, n_sc, chunk=m // 2, unroll=16)
    tcp = _tc_partial(pos, neg, n_tc, br=256, bc=4096)
    out = pl.pallas_call(
        functools.partial(_finish_kernel, n_total=n),
        out_shape=jax.ShapeDtypeStruct((1, 1), jnp.float32),
    )(pos[n_tc:], mv, sv, tcp)
    return out[0, 0]


# speedup vs baseline: 1.0630x; 1.0630x over previous
import functools

import jax
import jax.numpy as jnp
from jax import lax
from jax.experimental import pallas as pl
from jax.experimental.pallas import tpu as pltpu
from jax.experimental.pallas import tpu_sc as plsc

_INV_T = 10.0
_NEG_INIT = -1e30


def _sc_rowstats(neg, row_off, n_sc, chunk, unroll):
    _, m = neg.shape
    info = plsc.get_sparse_core_info()
    ncores, nsub, lanes = info.num_cores, info.num_subcores, info.num_lanes
    nw = ncores * nsub
    rows_w = n_sc // nw
    assert rows_w * nw == n_sc and m // chunk == 2 and m % chunk == 0
    nacc = 8

    mesh = plsc.VectorSubcoreMesh(core_axis_name="c", subcore_axis_name="s")

    @functools.partial(
        pl.kernel,
        mesh=mesh,
        out_type=(
            jax.ShapeDtypeStruct((n_sc, lanes), jnp.float32),
            jax.ShapeDtypeStruct((n_sc, lanes), jnp.float32),
        ),
        scratch_types=[
            pltpu.VMEM((chunk,), jnp.float32),
            pltpu.VMEM((chunk,), jnp.float32),
            pltpu.VMEM((rows_w, lanes), jnp.float32),
            pltpu.VMEM((rows_w, lanes), jnp.float32),
            pltpu.SemaphoreType.DMA,
            pltpu.SemaphoreType.DMA,
        ],
    )
    def k(neg_hbm, mv_hbm, sv_hbm, buf0, buf1, mrow, srow, sem0, sem1):
        wid = lax.axis_index("s") * ncores + lax.axis_index("c")
        base = row_off + wid * rows_w

        def process(buf, mv, sv):
            @plsc.parallel_loop(
                0, chunk, step=unroll * lanes,
                carry=tuple(
                    jnp.full((lanes,), _NEG_INIT, jnp.float32) for _ in range(nacc)
                ),
            )
            def max_loop(off, cms):
                cms = list(cms)
                for u in range(unroll):
                    x = buf[pl.ds(off + u * lanes, lanes)]
                    cms[u % nacc] = jnp.maximum(cms[u % nacc], x)
                return tuple(cms)

            cm = max_loop[0]
            for a in range(1, nacc):
                cm = jnp.maximum(cm, max_loop[a])
            m_new = jnp.maximum(mv, cm * _INV_T)

            @plsc.parallel_loop(
                0, chunk, step=unroll * lanes,
                carry=tuple(jnp.zeros((lanes,), jnp.float32) for _ in range(nacc)),
            )
            def sum_loop(off, accs):
                accs = list(accs)
                for u in range(unroll):
                    x = buf[pl.ds(off + u * lanes, lanes)]
                    accs[u % nacc] = accs[u % nacc] + jnp.exp(x * _INV_T - m_new)
                return tuple(accs)

            se = sum_loop[0]
            for a in range(1, nacc):
                se = se + sum_loop[a]
            sv = sv * jnp.exp(mv - m_new) + se
            return m_new, sv

        pltpu.make_async_copy(
            neg_hbm.at[base, pl.ds(0, chunk)], buf0, sem0
        ).start()

        def row_body(r, carry):
            row = base + r
            pltpu.make_async_copy(
                neg_hbm.at[row, pl.ds(0, chunk)], buf0, sem0
            ).wait()
            pltpu.make_async_copy(
                neg_hbm.at[row, pl.ds(chunk, chunk)], buf1, sem1
            ).start()
            mv0 = jnp.full((lanes,), _NEG_INIT, jnp.float32)
            sv0 = jnp.zeros((lanes,), jnp.float32)
            mv, sv = process(buf0, mv0, sv0)
            pltpu.make_async_copy(
                neg_hbm.at[row, pl.ds(chunk, chunk)], buf1, sem1
            ).wait()

            @pl.when(r + 1 < rows_w)
            def _():
                pltpu.make_async_copy(
                    neg_hbm.at[row + 1, pl.ds(0, chunk)], buf0, sem0
                ).start()

            mv, sv = process(buf1, mv, sv)
            mrow[r, :] = mv
            srow[r, :] = sv
            return carry

        lax.fori_loop(0, rows_w, row_body, 0)
        pltpu.sync_copy(mrow, mv_hbm.at[pl.ds(wid * rows_w, rows_w)])
        pltpu.sync_copy(srow, sv_hbm.at[pl.ds(wid * rows_w, rows_w)])

    return k(neg)


def _tc_partial_kernel(pos_ref, neg_ref, out_ref, m_ref, s_ref, acc_ref, *, nr, nc):
    ri = pl.program_id(0)
    ci = pl.program_id(1)

    @pl.when(jnp.logical_and(ri == 0, ci == 0))
    def _():
        acc_ref[0, 0] = 0.0

    p = pos_ref[:, :] * _INV_T

    @pl.when(ci == 0)
    def _():
        m_ref[:, :] = p
        s_ref[:, :] = jnp.ones_like(p)

    blk = neg_ref[:, :] * _INV_T
    bm = jnp.max(blk, axis=1, keepdims=True)
    m_old = m_ref[:, :]
    m_new = jnp.maximum(m_old, bm)
    s_ref[:, :] = s_ref[:, :] * jnp.exp(m_old - m_new) + jnp.sum(
        jnp.exp(blk - m_new), axis=1, keepdims=True
    )
    m_ref[:, :] = m_new

    @pl.when(ci == nc - 1)
    def _():
        lse = m_ref[:, :] + jnp.log(s_ref[:, :])
        acc_ref[0, 0] += jnp.sum(lse - p)

        @pl.when(ri == nr - 1)
        def _():
            out_ref[:, :] = jnp.full((1, 1), acc_ref[0, 0], jnp.float32)


def _tc_partial(pos, neg, n_tc, br, bc):
    _, m = neg.shape
    nr = n_tc // br
    nc = m // bc
    return pl.pallas_call(
        functools.partial(_tc_partial_kernel, nr=nr, nc=nc),
        grid=(nr, nc),
        in_specs=[
            pl.BlockSpec((br, 1), lambda ri, ci: (ri, 0)),
            pl.BlockSpec((br, bc), lambda ri, ci: (ri, ci)),
        ],
        out_specs=pl.BlockSpec((1, 1), lambda ri, ci: (0, 0)),
        out_shape=jax.ShapeDtypeStruct((1, 1), jnp.float32),
        scratch_shapes=[
            pltpu.VMEM((br, 1), jnp.float32),
            pltpu.VMEM((br, 1), jnp.float32),
            pltpu.SMEM((1, 1), jnp.float32),
        ],
        compiler_params=pltpu.CompilerParams(
            dimension_semantics=("arbitrary", "arbitrary"),
        ),
    )(pos, neg)


def _finish_kernel(pos_ref, mv_ref, sv_ref, tcp_ref, out_ref, *, n_total):
    p = pos_ref[:, :] * _INV_T
    mv = mv_ref[:, :]
    sv = sv_ref[:, :]
    mt = jnp.maximum(jnp.max(mv, axis=1, keepdims=True), p)
    s = jnp.sum(sv * jnp.exp(mv - mt), axis=1, keepdims=True) + jnp.exp(p - mt)
    lse = mt + jnp.log(s)
    total = jnp.sum(lse - p) + tcp_ref[0, 0]
    out_ref[:, :] = jnp.full((1, 1), total / n_total, jnp.float32)


def kernel(pos, neg):
    n, m = neg.shape
    n_tc = 2560
    n_sc = n - n_tc
    mv, sv = _sc_rowstats(neg, n_tc, n_sc, chunk=m // 2, unroll=32)
    tcp = _tc_partial(pos, neg, n_tc, br=256, bc=4096)
    out = pl.pallas_call(
        functools.partial(_finish_kernel, n_total=n),
        out_shape=jax.ShapeDtypeStruct((1, 1), jnp.float32),
    )(pos[n_tc:], mv, sv, tcp)
    return out[0, 0]
